# Initial kernel scaffold; baseline (speedup 1.0000x reference)
import jax
import jax.numpy as jnp
from jax.experimental import pallas as pl


def kernel(x, edge_index, Wl0, Wr0, b0, Wl1, Wr1, b1, prelu_a, W_ih, W_hh, b_ih, b_hh):
    raise NotImplementedError("write your pallas kernel here")


# speedup vs baseline: 1.7462x; 1.7462x over previous
import functools

import jax
import jax.numpy as jnp
from jax import lax
from jax.experimental import pallas as pl
from jax.experimental.pallas import tpu as pltpu
from jax.experimental.pallas import tpu_sc as plsc

T = 12
N = 10000
E = 160000
H = 256
HALF = 128
SAMPLE_NUM = 4
TIMESPAN = 3

NC = 2
NS = 16
K = 128
NBLK = 80
EPT = NBLK * K
EPAD = NS * EPT
AGG_ROWS = 10008
CP_A = 632
CP_LAST = N - (NS - 1) * CP_A
ZLAST = AGG_ROWS - (NS - 1) * CP_A


def _sc_agg_body(with_deg, *refs):
    if with_deg:
        (y_hbm, srcp, dstp, ones128, zeros128,
         out_hbm, deg_hbm,
         agg_s, src_c, dst_c, idxb, stage,
         sem_g, sem_s) = refs
    else:
        (y_hbm, srcp, dstp, zeros128,
         out_hbm,
         agg_s, src_c, dst_c, idxb, stage,
         sem_g, sem_s) = refs

    c = lax.axis_index("c")
    s = lax.axis_index("s")

    pltpu.sync_copy(srcp.at[s], src_c)
    pltpu.sync_copy(dstp.at[s], dst_c)

    def tile_range(fn_a, fn_last):
        @pl.when(s < NS - 1)
        def _():
            fn_a(pl.multiple_of(s * CP_A, 8))

        @pl.when(s == NS - 1)
        def _():
            fn_last((NS - 1) * CP_A)

    def zero_rows():
        def z(off, sz):
            nchunks = sz // K
            for jz in range(nchunks):
                pltpu.sync_copy(zeros128, agg_s.at[pl.ds(off + jz * K, K)])
            tail = sz - nchunks * K
            if tail:
                pltpu.sync_copy(zeros128.at[pl.ds(0, tail)],
                                agg_s.at[pl.ds(off + nchunks * K, tail)])
        tile_range(lambda off: z(off, CP_A), lambda off: z(off, ZLAST))

    def copy_out(make_dst):
        tile_range(
            lambda off: pltpu.sync_copy(agg_s.at[pl.ds(off, CP_A)],
                                        make_dst(off, CP_A)),
            lambda off: pltpu.sync_copy(agg_s.at[pl.ds(off, CP_LAST)],
                                        make_dst(off, CP_LAST)))

    def scatter_pass(j, data_src):
        pltpu.async_copy(data_src, agg_s.at[dst_c.at[j]], sem_s,
                         add=True).wait()

    if with_deg:
        zero_rows()
        pltpu.sync_copy(ones128, stage)
        plsc.subcore_barrier()

        def deg_blk(j, _):
            scatter_pass(j, stage)
            return _
        lax.fori_loop(0, NBLK, deg_blk, None)
        plsc.subcore_barrier()

        @pl.when(c == 0)
        def _():
            copy_out(lambda off, sz: deg_hbm.at[pl.ds(off, sz)])
        plsc.subcore_barrier()

    def body_t(t, _):
        zero_rows()
        plsc.subcore_barrier()

        base = (c * T + t) * N

        def body_j(j, _):
            row = src_c.at[j]
            for u in range(K // 16):
                sl = pl.ds(u * 16, 16)
                idxb[sl] = row[sl] + base
            pltpu.async_copy(y_hbm.at[idxb], stage, sem_g).wait()
            scatter_pass(j, stage)
            return _
        lax.fori_loop(0, NBLK, body_j, None)
        plsc.subcore_barrier()

        copy_out(lambda off, sz: out_hbm.at[c, t, pl.ds(off, sz)])
        plsc.subcore_barrier()
        return _
    lax.fori_loop(0, T, body_t, None)


def _make_sc_agg(with_deg):
    mesh = plsc.VectorSubcoreMesh(core_axis_name="c", subcore_axis_name="s")
    out_type = [jax.ShapeDtypeStruct((NC, T, N, HALF), jnp.float32)]
    if with_deg:
        out_type.append(jax.ShapeDtypeStruct((N, HALF), jnp.float32))
    scratch = [
        pltpu.VMEM_SHARED((AGG_ROWS, HALF), jnp.float32),
        pltpu.VMEM((NBLK, K), jnp.int32),
        pltpu.VMEM((NBLK, K), jnp.int32),
        pltpu.VMEM((K,), jnp.int32),
        pltpu.VMEM((K, HALF), jnp.float32),
        pltpu.SemaphoreType.DMA, pltpu.SemaphoreType.DMA,
    ]
    return pl.kernel(
        functools.partial(_sc_agg_body, with_deg),
        mesh=mesh,
        out_type=tuple(out_type) if len(out_type) > 1 else out_type[0],
        scratch_types=scratch,
    )


BN = 2000
NI = N // BN


def _dotT(a, w):
    return lax.dot_general(a, w, (((1,), (1,)), ((), ())),
                           preferred_element_type=jnp.float32)


def _tc_proj_body(x_ref, w_ref, out_ref):
    y = _dotT(x_ref[0], w_ref[...])
    out_ref[0, 0] = y[:, :HALF]
    out_ref[1, 0] = y[:, HALF:]


def _tc_proj(x, w):
    return pl.pallas_call(
        _tc_proj_body,
        grid=(T, NI),
        in_specs=[
            pl.BlockSpec((1, BN, H), lambda t, i: (t, i, 0)),
            pl.BlockSpec((H, H), lambda t, i: (0, 0)),
        ],
        out_specs=pl.BlockSpec((NC, 1, BN, HALF), lambda t, i: (0, t, i, 0)),
        out_shape=jax.ShapeDtypeStruct((NC, T, N, HALF), jnp.float32),
    )(x, w)


def _tc_mid_body(a0_ref, a1_ref, deg_ref, x_ref, wr0_ref, wl1_ref, wr1_ref,
                 b0_ref, pa_ref, y1_ref, r1_ref):
    rdeg = 1.0 / jnp.maximum(deg_ref[:, 0:1], 1.0)
    mean = jnp.concatenate([a0_ref[0, 0], a1_ref[0, 0]], axis=-1) * rdeg
    h1 = mean + _dotT(x_ref[0], wr0_ref[...]) + b0_ref[...]
    h1 = jnp.where(h1 > 0, h1, pa_ref[...] * h1)
    y1 = _dotT(h1, wl1_ref[...])
    y1_ref[0, 0] = y1[:, :HALF]
    y1_ref[1, 0] = y1[:, HALF:]
    r1_ref[0] = _dotT(h1, wr1_ref[...])


def _tc_mid(agg0, deg16, x, Wr0, Wl1, Wr1, b0, prelu_a):
    return pl.pallas_call(
        _tc_mid_body,
        grid=(T, NI),
        in_specs=[
            pl.BlockSpec((1, 1, BN, HALF), lambda t, i: (0, t, i, 0)),
            pl.BlockSpec((1, 1, BN, HALF), lambda t, i: (1, t, i, 0)),
            pl.BlockSpec((BN, HALF), lambda t, i: (i, 0)),
            pl.BlockSpec((1, BN, H), lambda t, i: (t, i, 0)),
            pl.BlockSpec((H, H), lambda t, i: (0, 0)),
            pl.BlockSpec((H, H), lambda t, i: (0, 0)),
            pl.BlockSpec((H, H), lambda t, i: (0, 0)),
            pl.BlockSpec((1, H), lambda t, i: (0, 0)),
            pl.BlockSpec((1, H), lambda t, i: (0, 0)),
        ],
        out_specs=[
            pl.BlockSpec((NC, 1, BN, HALF), lambda t, i: (0, t, i, 0)),
            pl.BlockSpec((1, BN, H), lambda t, i: (t, i, 0)),
        ],
        out_shape=[
            jax.ShapeDtypeStruct((NC, T, N, HALF), jnp.float32),
            jax.ShapeDtypeStruct((T, N, H), jnp.float32),
        ],
    )(agg0, agg0, deg16, x, Wr0, Wl1, Wr1, b0, prelu_a)


def _tc_pool_body(a0_ref, a1_ref, deg_ref, r1_ref, b1_ref, out_ref):
    i = pl.program_id(1)
    rdeg = 1.0 / jnp.maximum(deg_ref[:, 0:1], 1.0)
    h2 = (jnp.concatenate([a0_ref[0, 0], a1_ref[0, 0]], axis=-1) * rdeg
          + r1_ref[0] + b1_ref[...])
    m = jnp.max(h2, axis=0, keepdims=True)[None]

    @pl.when(i == 0)
    def _():
        out_ref[...] = m

    @pl.when(i > 0)
    def _():
        out_ref[...] = jnp.maximum(out_ref[...], m)


def _tc_pool(agg1, deg16, r1, b1):
    return pl.pallas_call(
        _tc_pool_body,
        grid=(T, NI),
        in_specs=[
            pl.BlockSpec((1, 1, BN, HALF), lambda t, i: (0, t, i, 0)),
            pl.BlockSpec((1, 1, BN, HALF), lambda t, i: (1, t, i, 0)),
            pl.BlockSpec((BN, HALF), lambda t, i: (i, 0)),
            pl.BlockSpec((1, BN, H), lambda t, i: (t, i, 0)),
            pl.BlockSpec((1, H), lambda t, i: (0, 0)),
        ],
        out_specs=pl.BlockSpec((1, 1, H), lambda t, i: (t, 0, 0)),
        out_shape=jax.ShapeDtypeStruct((T, 1, H), jnp.float32),
    )(agg1, agg1, deg16, r1, b1)


def _tc_head_body(emb_ref, wih_ref, whh_ref, bih_ref, bhh_ref,
                  nce_ref, acc_ref, z_ref):
    def step(tt, h):
        x_t = emb_ref[pl.ds(tt, 1), :]
        gi = _dotT(x_t, wih_ref[...]) + bih_ref[...]
        gh = _dotT(h, whh_ref[...]) + bhh_ref[...]
        r = jax.nn.sigmoid(gi[:, :H] + gh[:, :H])
        zg = jax.nn.sigmoid(gi[:, H:2 * H] + gh[:, H:2 * H])
        n = jnp.tanh(gi[:, 2 * H:] + r * gh[:, 2 * H:])
        h2 = (1.0 - zg) * n + zg * h
        z_ref[pl.ds(tt, 1), :] = h2
        return h2
    lax.fori_loop(0, T, step, jnp.zeros((1, H), jnp.float32))

    neg_dist = T // 6
    end = T - SAMPLE_NUM - neg_dist - TIMESPAN + 2
    start = T // 8 if T // 8 < end else 0
    cnt = end - start

    nce = jnp.zeros((1, 1), jnp.float32)
    correct = jnp.zeros((1, 1), jnp.float32)
    for t_sample in range(start, end):
        c_t = z_ref[pl.ds(t_sample, 1), :]
        cnorm = jnp.sqrt(jnp.sum(c_t * c_t))
        for i in range(1, TIMESPAN + 1):
            idxs = [t_sample + i] + [t_sample + i + neg_dist + n - 1
                                     for n in range(1, SAMPLE_NUM)]
            samples = jnp.concatenate(
                [emb_ref[pl.ds(ix, 1), :] for ix in idxs], axis=0)
            dots = _dotT(samples, c_t)
            norms = jnp.sqrt(jnp.sum(samples * samples, axis=1, keepdims=True))
            total = dots / jnp.maximum(norms * cnorm, 1e-8)
            mx = jnp.max(total)
            lse = jnp.log(jnp.sum(jnp.exp(total - mx)))
            nce = nce + (total[0:1, :] - mx - lse)
            others = jnp.max(total[1:, :])
            correct = correct + jnp.where(total[0:1, :] >= others, 1.0, 0.0)
    nce_ref[...] = nce / (-1.0 * cnt * TIMESPAN)
    acc_ref[...] = correct / (cnt * TIMESPAN)


def _tc_head(emb, W_ih, W_hh, b_ih, b_hh):
    return pl.pallas_call(
        _tc_head_body,
        in_specs=[
            pl.BlockSpec((T, H), lambda: (0, 0)),
            pl.BlockSpec((3 * H, H), lambda: (0, 0)),
            pl.BlockSpec((3 * H, H), lambda: (0, 0)),
            pl.BlockSpec((1, 3 * H), lambda: (0, 0)),
            pl.BlockSpec((1, 3 * H), lambda: (0, 0)),
        ],
        out_specs=[
            pl.BlockSpec((1, 1), lambda: (0, 0)),
            pl.BlockSpec((1, 1), lambda: (0, 0)),
        ],
        out_shape=[
            jax.ShapeDtypeStruct((1, 1), jnp.float32),
            jax.ShapeDtypeStruct((1, 1), jnp.float32),
        ],
        scratch_shapes=[pltpu.VMEM((T, H), jnp.float32)],
    )(emb, W_ih, W_hh, b_ih, b_hh)


def kernel(x, edge_index, Wl0, Wr0, b0, Wl1, Wr1, b1, prelu_a,
           W_ih, W_hh, b_ih, b_hh):
    src = edge_index[0]
    dst = edge_index[1]
    pad = EPAD - E
    srcp = jnp.concatenate([src, jnp.zeros((pad,), jnp.int32)]).reshape(NS, NBLK, K)
    dstp = jnp.concatenate([dst, jnp.full((pad,), N, jnp.int32)]).reshape(NS, NBLK, K)

    ones128 = jnp.ones((K, HALF), jnp.float32)
    zeros128 = jnp.zeros((K, HALF), jnp.float32)

    b0r = b0.reshape(1, H)
    b1r = b1.reshape(1, H)
    par = prelu_a.reshape(1, H)
    bihr = b_ih.reshape(1, 3 * H)
    bhhr = b_hh.reshape(1, 3 * H)

    y0 = _tc_proj(x, Wl0)
    agg0, deg = _make_sc_agg(True)(
        y0.reshape(NC * T * N, HALF), srcp, dstp, ones128, zeros128)
    y1, r1 = _tc_mid(agg0, deg, x, Wr0, Wl1, Wr1, b0r, par)
    agg1 = _make_sc_agg(False)(y1.reshape(NC * T * N, HALF), srcp, dstp,
                               zeros128)
    emb = _tc_pool(agg1, deg, r1, b1r).reshape(T, H)
    nce, acc = _tc_head(emb, W_ih, W_hh, bihr, bhhr)
    return (nce.reshape(()), acc.reshape(()))


# speedup vs baseline: 4.6084x; 2.6392x over previous
import functools

import jax
import jax.numpy as jnp
from jax import lax
from jax.experimental import pallas as pl
from jax.experimental.pallas import tpu as pltpu
from jax.experimental.pallas import tpu_sc as plsc

T = 12
N = 10000
E = 160000
H = 256
HALF = 128
SAMPLE_NUM = 4
TIMESPAN = 3

NC = 2
NS = 16
K = 88
NBLK = 114
NBUF = 3
EPT = NBLK * K
EPAD = NS * EPT
AGG_ROWS = 10008
CP_A = 632
CP_LAST = N - (NS - 1) * CP_A
ZLAST = AGG_ROWS - (NS - 1) * CP_A


def _sc_agg_body(with_deg, *refs):
    if with_deg:
        (y_hbm, srcp, dstp, ones128, zeros128,
         out_hbm, deg_hbm,
         agg_s, idxall, didx, st0, st1, st2,
         sg0, sg1, sg2, ss0, ss1, ss2, si0, si1, si2) = refs
    else:
        (y_hbm, srcp, dstp, zeros128,
         out_hbm,
         agg_s, idxall, didx, st0, st1, st2,
         sg0, sg1, sg2, ss0, ss1, ss2, si0, si1, si2) = refs
    stage = (st0, st1, st2)
    sem_g = (sg0, sg1, sg2)
    sem_s = (ss0, ss1, ss2)
    sem_i = (si0, si1, si2)

    c = lax.axis_index("c")
    s = lax.axis_index("s")

    pltpu.sync_copy(srcp.at[s], idxall)

    def add_delta(d):
        def rowfn(jr, _):
            r = idxall.at[jr]
            for u in range(K // 16):
                sl = pl.ds(u * 16, 16)
                r[sl] = r[sl] + d
            return _
        lax.fori_loop(0, NBLK, rowfn, None)

    add_delta(c * (T * N) - N)

    def tile_range(fn_a, fn_last):
        @pl.when(s < NS - 1)
        def _():
            fn_a(pl.multiple_of(s * CP_A, 8))

        @pl.when(s == NS - 1)
        def _():
            fn_last((NS - 1) * CP_A)

    def zero_rows():
        def z(off, sz):
            nchunks = sz // K
            for jz in range(nchunks):
                pltpu.sync_copy(zeros128, agg_s.at[pl.ds(off + jz * K, K)])
            tail = sz - nchunks * K
            if tail:
                pltpu.sync_copy(zeros128.at[pl.ds(0, tail)],
                                agg_s.at[pl.ds(off + nchunks * K, tail)])
        tile_range(lambda off: z(off, CP_A), lambda off: z(off, ZLAST))

    def copy_out(make_dst):
        tile_range(
            lambda off: pltpu.sync_copy(agg_s.at[pl.ds(off, CP_A)],
                                        make_dst(off, CP_A)),
            lambda off: pltpu.sync_copy(agg_s.at[pl.ds(off, CP_LAST)],
                                        make_dst(off, CP_LAST)))

    def didx_load(m, b):
        return pltpu.async_copy(dstp.at[s, m], didx.at[b], sem_i[b])

    def gather(m, b):
        return pltpu.async_copy(y_hbm.at[idxall.at[m]], stage[b], sem_g[b])

    def scatter(b):
        return pltpu.async_copy(stage[b], agg_s.at[didx.at[b]], sem_s[b],
                                add=True)

    def wait_scatter(b):
        pltpu.make_async_copy(stage[b], agg_s.at[didx.at[b]], sem_s[b]).wait()

    if with_deg:
        zero_rows()
        pltpu.sync_copy(ones128, st0)
        for b in range(2):
            didx_load(b, b)
        plsc.subcore_barrier()

        def deg_tri(jj, _):
            for b in range(NBUF):
                m = jj * NBUF + b
                pltpu.make_async_copy(dstp.at[s, m], didx.at[b],
                                      sem_i[b]).wait()
                pltpu.async_copy(st0, agg_s.at[didx.at[b]], sem_s[b],
                                 add=True)
                bf = (b + 2) % NBUF

                @pl.when(m >= 1)
                def _():
                    pltpu.make_async_copy(st0, agg_s.at[didx.at[bf]],
                                          sem_s[bf]).wait()

                @pl.when(m + 2 < NBLK)
                def _():
                    didx_load(m + 2, bf)
            return _
        lax.fori_loop(0, NBLK // NBUF, deg_tri, None)
        pltpu.make_async_copy(st0, agg_s.at[didx.at[2]], sem_s[2]).wait()
        plsc.subcore_barrier()

        @pl.when(c == 0)
        def _():
            copy_out(lambda off, sz: deg_hbm.at[pl.ds(off, sz)])
        plsc.subcore_barrier()

    def body_t(t, _):
        add_delta(N)
        for b in range(2):
            didx_load(b, b)
            gather(b, b)
        zero_rows()
        plsc.subcore_barrier()

        def tri(jj, _):
            for b in range(NBUF):
                m = jj * NBUF + b
                pltpu.make_async_copy(y_hbm.at[idxall.at[m]], stage[b],
                                      sem_g[b]).wait()
                pltpu.make_async_copy(dstp.at[s, m], didx.at[b],
                                      sem_i[b]).wait()
                scatter(b)
                bf = (b + 2) % NBUF

                @pl.when(m >= 1)
                def _():
                    wait_scatter(bf)

                @pl.when(m + 2 < NBLK)
                def _():
                    didx_load(m + 2, bf)
                    gather(m + 2, bf)
            return _
        lax.fori_loop(0, NBLK // NBUF, tri, None)
        wait_scatter(2)
        plsc.subcore_barrier()

        copy_out(lambda off, sz: out_hbm.at[c, t, pl.ds(off, sz)])
        plsc.subcore_barrier()
        return _
    lax.fori_loop(0, T, body_t, None)


def _make_sc_agg(with_deg):
    mesh = plsc.VectorSubcoreMesh(core_axis_name="c", subcore_axis_name="s")
    out_type = [jax.ShapeDtypeStruct((NC, T, N, HALF), jnp.float32)]
    if with_deg:
        out_type.append(jax.ShapeDtypeStruct((N, HALF), jnp.float32))
    scratch = [
        pltpu.VMEM_SHARED((AGG_ROWS, HALF), jnp.float32),
        pltpu.VMEM((NBLK, K), jnp.int32),
        pltpu.VMEM((NBUF, K), jnp.int32),
        pltpu.VMEM((K, HALF), jnp.float32),
        pltpu.VMEM((K, HALF), jnp.float32),
        pltpu.VMEM((K, HALF), jnp.float32),
    ] + [pltpu.SemaphoreType.DMA] * 9
    return pl.kernel(
        functools.partial(_sc_agg_body, with_deg),
        mesh=mesh,
        out_type=tuple(out_type) if len(out_type) > 1 else out_type[0],
        scratch_types=scratch,
    )


BN = 2000
NI = N // BN


def _dotT(a, w):
    return lax.dot_general(a, w, (((1,), (1,)), ((), ())),
                           preferred_element_type=jnp.float32)


def _tc_proj_body(x_ref, w_ref, out_ref):
    y = _dotT(x_ref[0], w_ref[...])
    out_ref[0, 0] = y[:, :HALF]
    out_ref[1, 0] = y[:, HALF:]


def _tc_proj(x, w):
    return pl.pallas_call(
        _tc_proj_body,
        grid=(T, NI),
        in_specs=[
            pl.BlockSpec((1, BN, H), lambda t, i: (t, i, 0)),
            pl.BlockSpec((H, H), lambda t, i: (0, 0)),
        ],
        out_specs=pl.BlockSpec((NC, 1, BN, HALF), lambda t, i: (0, t, i, 0)),
        out_shape=jax.ShapeDtypeStruct((NC, T, N, HALF), jnp.float32),
    )(x, w)


def _tc_mid_body(a0_ref, a1_ref, deg_ref, x_ref, wr0_ref, wl1_ref, wr1_ref,
                 b0_ref, pa_ref, y1_ref, r1_ref):
    rdeg = 1.0 / jnp.maximum(deg_ref[:, 0:1], 1.0)
    mean = jnp.concatenate([a0_ref[0, 0], a1_ref[0, 0]], axis=-1) * rdeg
    h1 = mean + _dotT(x_ref[0], wr0_ref[...]) + b0_ref[...]
    h1 = jnp.where(h1 > 0, h1, pa_ref[...] * h1)
    y1 = _dotT(h1, wl1_ref[...])
    y1_ref[0, 0] = y1[:, :HALF]
    y1_ref[1, 0] = y1[:, HALF:]
    r1_ref[0] = _dotT(h1, wr1_ref[...])


def _tc_mid(agg0, deg16, x, Wr0, Wl1, Wr1, b0, prelu_a):
    return pl.pallas_call(
        _tc_mid_body,
        grid=(T, NI),
        in_specs=[
            pl.BlockSpec((1, 1, BN, HALF), lambda t, i: (0, t, i, 0)),
            pl.BlockSpec((1, 1, BN, HALF), lambda t, i: (1, t, i, 0)),
            pl.BlockSpec((BN, HALF), lambda t, i: (i, 0)),
            pl.BlockSpec((1, BN, H), lambda t, i: (t, i, 0)),
            pl.BlockSpec((H, H), lambda t, i: (0, 0)),
            pl.BlockSpec((H, H), lambda t, i: (0, 0)),
            pl.BlockSpec((H, H), lambda t, i: (0, 0)),
            pl.BlockSpec((1, H), lambda t, i: (0, 0)),
            pl.BlockSpec((1, H), lambda t, i: (0, 0)),
        ],
        out_specs=[
            pl.BlockSpec((NC, 1, BN, HALF), lambda t, i: (0, t, i, 0)),
            pl.BlockSpec((1, BN, H), lambda t, i: (t, i, 0)),
        ],
        out_shape=[
            jax.ShapeDtypeStruct((NC, T, N, HALF), jnp.float32),
            jax.ShapeDtypeStruct((T, N, H), jnp.float32),
        ],
    )(agg0, agg0, deg16, x, Wr0, Wl1, Wr1, b0, prelu_a)


def _tc_pool_body(a0_ref, a1_ref, deg_ref, r1_ref, b1_ref, out_ref):
    i = pl.program_id(1)
    rdeg = 1.0 / jnp.maximum(deg_ref[:, 0:1], 1.0)
    h2 = (jnp.concatenate([a0_ref[0, 0], a1_ref[0, 0]], axis=-1) * rdeg
          + r1_ref[0] + b1_ref[...])
    m = jnp.max(h2, axis=0, keepdims=True)[None]

    @pl.when(i == 0)
    def _():
        out_ref[...] = m

    @pl.when(i > 0)
    def _():
        out_ref[...] = jnp.maximum(out_ref[...], m)


def _tc_pool(agg1, deg16, r1, b1):
    return pl.pallas_call(
        _tc_pool_body,
        grid=(T, NI),
        in_specs=[
            pl.BlockSpec((1, 1, BN, HALF), lambda t, i: (0, t, i, 0)),
            pl.BlockSpec((1, 1, BN, HALF), lambda t, i: (1, t, i, 0)),
            pl.BlockSpec((BN, HALF), lambda t, i: (i, 0)),
            pl.BlockSpec((1, BN, H), lambda t, i: (t, i, 0)),
            pl.BlockSpec((1, H), lambda t, i: (0, 0)),
        ],
        out_specs=pl.BlockSpec((1, 1, H), lambda t, i: (t, 0, 0)),
        out_shape=jax.ShapeDtypeStruct((T, 1, H), jnp.float32),
    )(agg1, agg1, deg16, r1, b1)


def _tc_head_body(emb_ref, wih_ref, whh_ref, bih_ref, bhh_ref,
                  nce_ref, acc_ref, z_ref):
    def step(tt, h):
        x_t = emb_ref[pl.ds(tt, 1), :]
        gi = _dotT(x_t, wih_ref[...]) + bih_ref[...]
        gh = _dotT(h, whh_ref[...]) + bhh_ref[...]
        r = jax.nn.sigmoid(gi[:, :H] + gh[:, :H])
        zg = jax.nn.sigmoid(gi[:, H:2 * H] + gh[:, H:2 * H])
        n = jnp.tanh(gi[:, 2 * H:] + r * gh[:, 2 * H:])
        h2 = (1.0 - zg) * n + zg * h
        z_ref[pl.ds(tt, 1), :] = h2
        return h2
    lax.fori_loop(0, T, step, jnp.zeros((1, H), jnp.float32))

    neg_dist = T // 6
    end = T - SAMPLE_NUM - neg_dist - TIMESPAN + 2
    start = T // 8 if T // 8 < end else 0
    cnt = end - start

    nce = jnp.zeros((1, 1), jnp.float32)
    correct = jnp.zeros((1, 1), jnp.float32)
    for t_sample in range(start, end):
        c_t = z_ref[pl.ds(t_sample, 1), :]
        cnorm = jnp.sqrt(jnp.sum(c_t * c_t))
        for i in range(1, TIMESPAN + 1):
            idxs = [t_sample + i] + [t_sample + i + neg_dist + n - 1
                                     for n in range(1, SAMPLE_NUM)]
            samples = jnp.concatenate(
                [emb_ref[pl.ds(ix, 1), :] for ix in idxs], axis=0)
            dots = _dotT(samples, c_t)
            norms = jnp.sqrt(jnp.sum(samples * samples, axis=1, keepdims=True))
            total = dots / jnp.maximum(norms * cnorm, 1e-8)
            mx = jnp.max(total)
            lse = jnp.log(jnp.sum(jnp.exp(total - mx)))
            nce = nce + (total[0:1, :] - mx - lse)
            others = jnp.max(total[1:, :])
            correct = correct + jnp.where(total[0:1, :] >= others, 1.0, 0.0)
    nce_ref[...] = nce / (-1.0 * cnt * TIMESPAN)
    acc_ref[...] = correct / (cnt * TIMESPAN)


def _tc_head(emb, W_ih, W_hh, b_ih, b_hh):
    return pl.pallas_call(
        _tc_head_body,
        in_specs=[
            pl.BlockSpec((T, H), lambda: (0, 0)),
            pl.BlockSpec((3 * H, H), lambda: (0, 0)),
            pl.BlockSpec((3 * H, H), lambda: (0, 0)),
            pl.BlockSpec((1, 3 * H), lambda: (0, 0)),
            pl.BlockSpec((1, 3 * H), lambda: (0, 0)),
        ],
        out_specs=[
            pl.BlockSpec((1, 1), lambda: (0, 0)),
            pl.BlockSpec((1, 1), lambda: (0, 0)),
        ],
        out_shape=[
            jax.ShapeDtypeStruct((1, 1), jnp.float32),
            jax.ShapeDtypeStruct((1, 1), jnp.float32),
        ],
        scratch_shapes=[pltpu.VMEM((T, H), jnp.float32)],
    )(emb, W_ih, W_hh, b_ih, b_hh)


def kernel(x, edge_index, Wl0, Wr0, b0, Wl1, Wr1, b1, prelu_a,
           W_ih, W_hh, b_ih, b_hh):
    src = edge_index[0]
    dst = edge_index[1]
    pad = EPAD - E
    srcp = jnp.concatenate([src, jnp.zeros((pad,), jnp.int32)]).reshape(NS, NBLK, K)
    dstp = jnp.concatenate([dst, jnp.full((pad,), N, jnp.int32)]).reshape(NS, NBLK, K)

    ones128 = jnp.ones((K, HALF), jnp.float32)
    zeros128 = jnp.zeros((K, HALF), jnp.float32)

    b0r = b0.reshape(1, H)
    b1r = b1.reshape(1, H)
    par = prelu_a.reshape(1, H)
    bihr = b_ih.reshape(1, 3 * H)
    bhhr = b_hh.reshape(1, 3 * H)

    y0 = _tc_proj(x, Wl0)
    agg0, deg = _make_sc_agg(True)(
        y0.reshape(NC * T * N, HALF), srcp, dstp, ones128, zeros128)
    y1, r1 = _tc_mid(agg0, deg, x, Wr0, Wl1, Wr1, b0r, par)
    agg1 = _make_sc_agg(False)(y1.reshape(NC * T * N, HALF), srcp, dstp,
                               zeros128)
    emb = _tc_pool(agg1, deg, r1, b1r).reshape(T, H)
    nce, acc = _tc_head(emb, W_ih, W_hh, bihr, bhhr)
    return (nce.reshape(()), acc.reshape(()))
